# baseline (speedup 1.0000x reference)
import functools

import jax
import jax.numpy as jnp
from jax import lax
from jax.experimental import pallas as pl
from jax.experimental.pallas import tpu as pltpu
from jax.experimental.pallas import tpu_sc as plsc

VOCAB = 1000000
TCB = 512
TC_STEPS = 977
HALF = TCB * TC_STEPS
MAX_LEN = 200
EMBED_DIM = 64
BATCH = 4096

NC = 2
NS = 16
NW = NC * NS

B_PER_W = BATCH // NW
SEQ_PER_CHUNK = 2
CHUNK = SEQ_PER_CHUNK * MAX_LEN
N_CHUNKS = B_PER_W // SEQ_PER_CHUNK
LANES = 16
VPR = EMBED_DIM // LANES
L_STARTS = tuple(range(0, MAX_LEN - LANES, LANES)) + (MAX_LEN - LANES,)

def _repack_body(lo_ref, hi_ref, out_ref):
    out_ref[:, :EMBED_DIM] = lo_ref[...].T
    out_ref[:, EMBED_DIM:] = hi_ref[...].T


_repack = pl.pallas_call(
    _repack_body,
    grid=(TC_STEPS,),
    in_specs=[
        pl.BlockSpec((EMBED_DIM, TCB), lambda i: (0, i)),
        pl.BlockSpec((EMBED_DIM, TCB), lambda i: (0, i + TC_STEPS)),
    ],
    out_specs=pl.BlockSpec((TCB, 2 * EMBED_DIM), lambda i: (i, 0)),
    out_shape=jax.ShapeDtypeStruct((HALF, 2 * EMBED_DIM), jnp.float32),
)


@functools.partial(
    pl.kernel,
    mesh=plsc.VectorSubcoreMesh(core_axis_name="c", subcore_axis_name="s"),
    out_type=jax.ShapeDtypeStruct((BATCH, MAX_LEN, EMBED_DIM), jnp.float32),
    scratch_types=[
        pltpu.VMEM((MAX_LEN, EMBED_DIM), jnp.float32),
        pltpu.VMEM((CHUNK,), jnp.int32),
        pltpu.VMEM((CHUNK,), jnp.int32),
        pltpu.VMEM((CHUNK, 2 * EMBED_DIM), jnp.float32),
        pltpu.VMEM((SEQ_PER_CHUNK, MAX_LEN, EMBED_DIM), jnp.float32),
        pltpu.SemaphoreType.DMA,
    ],
    compiler_params=pltpu.CompilerParams(use_tc_tiling_on_sc=True),
)
def _embed(idx_hbm, t2_hbm, pos_hbm, out_hbm,
           pos_v, sel_v, pair_v, rows_v, out_v, sem):
    wid = lax.axis_index("s") * NC + lax.axis_index("c")
    b_base = wid * B_PER_W
    pltpu.sync_copy(pos_hbm, pos_v)

    def chunk_body(ci, carry):
        bb = b_base + ci * SEQ_PER_CHUNK
        pltpu.sync_copy(idx_hbm.at[pl.ds(bb * MAX_LEN, CHUNK)], sel_v)

        def mk_pairs(v, c):
            sl = pl.ds(v * LANES, LANES)
            idx = sel_v[sl]
            ge = 1 - lax.shift_right_logical(idx - HALF, 31)
            pair_v[sl] = idx - HALF * ge
            sel_v[sl] = ge
            return c

        lax.fori_loop(0, CHUNK // LANES, mk_pairs, 0)
        pltpu.async_copy(t2_hbm.at[pair_v], rows_v, sem).wait()

        for s in range(SEQ_PER_CHUNK):
            for l0 in L_STARTS:
                par = sel_v[pl.ds(s * MAX_LEN + l0, LANES)]

                def row_k(k, c, s=s, l0=l0, par=par):
                    ksplat = lax.broadcast_in_dim(k, (LANES,), ())
                    pk = par.at[ksplat].get(mode="promise_in_bounds")
                    pf = pk.astype(jnp.float32)
                    r = s * MAX_LEN + l0 + k
                    for j in range(VPR):
                        lo = rows_v[r, pl.ds(j * LANES, LANES)]
                        hi = rows_v[r, pl.ds(EMBED_DIM + j * LANES, LANES)]
                        out_v[s, l0 + k, pl.ds(j * LANES, LANES)] = (
                            lo + pf * (hi - lo)
                            + pos_v[l0 + k, pl.ds(j * LANES, LANES)]
                        )
                    return c

                lax.fori_loop(0, LANES, row_k, 0)

        pltpu.sync_copy(out_v, out_hbm.at[pl.ds(bb, SEQ_PER_CHUNK)])
        return carry

    lax.fori_loop(0, N_CHUNKS, chunk_body, 0)


def kernel(inputs, token_table, pos_table):
    idx = inputs.reshape(-1).astype(jnp.int32)
    tt = token_table.T
    t2 = _repack(tt, tt)
    return _embed(idx, t2, pos_table)


# speedup vs baseline: 1.3261x; 1.3261x over previous
import functools

import jax
import jax.numpy as jnp
from jax import lax
from jax.experimental import pallas as pl
from jax.experimental.pallas import tpu as pltpu
from jax.experimental.pallas import tpu_sc as plsc

VOCAB = 1000000
MAX_LEN = 200
EMBED_DIM = 64
BATCH = 4096

NC = 2
NS = 16
NW = NC * NS

B_PER_W = BATCH // NW
SEQ_PER_CHUNK = 4
CHUNK = SEQ_PER_CHUNK * MAX_LEN
N_CHUNKS = B_PER_W // SEQ_PER_CHUNK
LANES = 16
VPR = EMBED_DIM // LANES


@functools.partial(
    pl.kernel,
    mesh=plsc.VectorSubcoreMesh(core_axis_name="c", subcore_axis_name="s"),
    out_type=jax.ShapeDtypeStruct((BATCH, MAX_LEN, EMBED_DIM), jnp.float32),
    scratch_types=[
        pltpu.VMEM((MAX_LEN, EMBED_DIM), jnp.float32),
        pltpu.VMEM((CHUNK,), jnp.int32),
        pltpu.VMEM((SEQ_PER_CHUNK, MAX_LEN, EMBED_DIM), jnp.float32),
        pltpu.SemaphoreType.DMA,
    ],
    compiler_params=pltpu.CompilerParams(use_tc_tiling_on_sc=False),
)
def _embed(idx_hbm, table_hbm, pos_hbm, out_hbm, pos_v, idx_v, rows_v, sem):
    wid = lax.axis_index("s") * NC + lax.axis_index("c")
    b_base = wid * B_PER_W
    pltpu.sync_copy(pos_hbm, pos_v)

    def chunk_body(ci, carry):
        bb = b_base + ci * SEQ_PER_CHUNK
        pltpu.sync_copy(idx_hbm.at[pl.ds(bb * MAX_LEN, CHUNK)], idx_v)
        copies = [
            pltpu.async_copy(
                table_hbm.at[idx_v.at[pl.ds(s * MAX_LEN, MAX_LEN)]],
                rows_v.at[s],
                sem,
            )
            for s in range(SEQ_PER_CHUNK)
        ]
        for cp in copies:
            cp.wait()

        def add_pos(l, c):
            for j in range(VPR):
                pv = pos_v[l, pl.ds(j * LANES, LANES)]
                for s in range(SEQ_PER_CHUNK):
                    rows_v[s, l, pl.ds(j * LANES, LANES)] = (
                        rows_v[s, l, pl.ds(j * LANES, LANES)] + pv
                    )
            return c

        lax.fori_loop(0, MAX_LEN, add_pos, 0)
        pltpu.sync_copy(rows_v, out_hbm.at[pl.ds(bb, SEQ_PER_CHUNK)])
        return carry

    lax.fori_loop(0, N_CHUNKS, chunk_body, 0)


def kernel(inputs, token_table, pos_table):
    idx = inputs.reshape(-1).astype(jnp.int32)
    return _embed(idx, token_table, pos_table)


# speedup vs baseline: 1.3523x; 1.0197x over previous
import functools

import jax
import jax.numpy as jnp
from jax import lax
from jax.experimental import pallas as pl
from jax.experimental.pallas import tpu as pltpu
from jax.experimental.pallas import tpu_sc as plsc

VOCAB = 1000000
MAX_LEN = 200
EMBED_DIM = 64
BATCH = 4096

NC = 2
NS = 16
NW = NC * NS

B_PER_W = BATCH // NW
SEQ_PER_CHUNK = 8
CHUNK = SEQ_PER_CHUNK * MAX_LEN
N_CHUNKS = B_PER_W // SEQ_PER_CHUNK
LANES = 16
VPR = EMBED_DIM // LANES


@functools.partial(
    pl.kernel,
    mesh=plsc.VectorSubcoreMesh(core_axis_name="c", subcore_axis_name="s"),
    out_type=jax.ShapeDtypeStruct((BATCH, MAX_LEN, EMBED_DIM), jnp.float32),
    scratch_types=[
        pltpu.VMEM((MAX_LEN, EMBED_DIM), jnp.float32),
        pltpu.VMEM((CHUNK,), jnp.int32),
        pltpu.VMEM((SEQ_PER_CHUNK, MAX_LEN, EMBED_DIM), jnp.float32),
        pltpu.SemaphoreType.DMA,
    ],
    compiler_params=pltpu.CompilerParams(use_tc_tiling_on_sc=False),
)
def _embed(idx_hbm, table_hbm, pos_hbm, out_hbm, pos_v, idx_v, rows_v, sem):
    wid = lax.axis_index("s") * NC + lax.axis_index("c")
    b_base = wid * B_PER_W
    pltpu.sync_copy(pos_hbm, pos_v)

    def chunk_body(ci, carry):
        bb = b_base + ci * SEQ_PER_CHUNK
        pltpu.sync_copy(idx_hbm.at[pl.ds(bb * MAX_LEN, CHUNK)], idx_v)
        copies = [
            pltpu.async_copy(
                table_hbm.at[idx_v.at[pl.ds(s * MAX_LEN, MAX_LEN)]],
                rows_v.at[s],
                sem,
            )
            for s in range(SEQ_PER_CHUNK)
        ]
        for cp in copies:
            cp.wait()

        def add_pos(l, c):
            for j in range(VPR):
                pv = pos_v[l, pl.ds(j * LANES, LANES)]
                for s in range(SEQ_PER_CHUNK):
                    rows_v[s, l, pl.ds(j * LANES, LANES)] = (
                        rows_v[s, l, pl.ds(j * LANES, LANES)] + pv
                    )
            return c

        lax.fori_loop(0, MAX_LEN, add_pos, 0)
        pltpu.sync_copy(rows_v, out_hbm.at[pl.ds(bb, SEQ_PER_CHUNK)])
        return carry

    lax.fori_loop(0, N_CHUNKS, chunk_body, 0)


def kernel(inputs, token_table, pos_table):
    idx = inputs.reshape(-1).astype(jnp.int32)
    return _embed(idx, token_table, pos_table)
